# Initial kernel scaffold; baseline (speedup 1.0000x reference)
import jax
import jax.numpy as jnp
from jax.experimental import pallas as pl


def kernel(infeat, labelTpesudo, labelT, FeatureDA, k):
    raise NotImplementedError("write your pallas kernel here")


# speedup vs baseline: 56.7425x; 56.7425x over previous
import functools

import jax
import jax.numpy as jnp
from jax import lax
from jax.experimental import pallas as pl
from jax.experimental.pallas import tpu as pltpu
from jax.experimental.pallas import tpu_sc as plsc

ORISIZE = 256
KER = 32
STRIDE = 2
PSIZE = (ORISIZE - KER) // STRIDE + 1
PPAD = 128
PFLAT = PSIZE * PPAD
IMG = ORISIZE * ORISIZE


def _pool_tc_body(x_ref, out_ref):
    x0 = x_ref[0, 0]
    x1 = x_ref[0, 1]
    m = jnp.maximum(x0, x1)
    e0 = jnp.exp(x0 - m)
    e1 = jnp.exp(x1 - m)
    s = e0 + e1

    pr = lax.broadcasted_iota(jnp.int32, (PPAD, ORISIZE), 0)
    jr = lax.broadcasted_iota(jnp.int32, (PPAD, ORISIZE), 1)
    wr = ((jr >= STRIDE * pr) & (jr < STRIDE * pr + KER)).astype(jnp.float32)
    jc = lax.broadcasted_iota(jnp.int32, (ORISIZE, PPAD), 0)
    pc = lax.broadcasted_iota(jnp.int32, (ORISIZE, PPAD), 1)
    wc = ((jc >= STRIDE * pc) & (jc < STRIDE * pc + KER)
          & (pc < PSIZE)).astype(jnp.float32)

    inv = jnp.float32(1.0 / (KER * KER))
    for c, p in ((0, e0 / s), (1, e1 / s)):
        t = jnp.dot(p, wc, preferred_element_type=jnp.float32,
                    precision=lax.Precision.HIGHEST)
        g = jnp.dot(wr, t, preferred_element_type=jnp.float32,
                    precision=lax.Precision.HIGHEST)
        out_ref[0, c] = (g * inv)[:PSIZE, :]


def _pooled_grids(infeat):
    b = infeat.shape[0]
    return pl.pallas_call(
        _pool_tc_body,
        grid=(b,),
        in_specs=[pl.BlockSpec((1, 2, ORISIZE, ORISIZE), lambda i: (i, 0, 0, 0))],
        out_specs=pl.BlockSpec((1, 2, PSIZE, PPAD), lambda i: (i, 0, 0, 0)),
        out_shape=jax.ShapeDtypeStruct((b, 2, PSIZE, PPAD), jnp.float32),
    )(infeat)


def _sc_mask_body(pooled_hbm, karr_hbm, mask_hbm, coord_hbm, val_hbm,
                  pbuf, mbuf, kbuf, bufm, bufi, cstage, vstage):
    i = lax.axis_index("s") * 2 + lax.axis_index("c")

    lane = lax.iota(jnp.int32, 16)
    zeros16 = jnp.zeros((16,), jnp.float32)
    ones16 = jnp.ones((16,), jnp.float32)

    pltpu.sync_copy(karr_hbm, kbuf)
    kshiftv = kbuf[...].astype(jnp.int32)

    def _zf(j, carry):
        base = j * 128
        for t in range(8):
            mbuf[pl.ds(base + t * 16, 16)] = zeros16
        return carry
    lax.fori_loop(0, IMG // 128, _zf, 0)

    for c in range(2):
        pltpu.sync_copy(pooled_hbm.at[i, c], pbuf)

        def _scan(r, carry):
            mx, ix = carry
            for u in range(8):
                v = pbuf[pl.ds(r * PPAD + u * 16, 16)]
                col = u * 16 + lane
                if u == 7:
                    v = jnp.where(col < PSIZE, v, jnp.float32(-1.0))
                nflat = (-(r * PSIZE) - col).astype(jnp.float32)
                upd = v > mx
                mx = jnp.where(upd, v, mx)
                ix = jnp.where(upd, nflat, ix)
            return mx, ix
        m0 = jnp.full((16,), -jnp.inf, jnp.float32)
        i0 = jnp.full((16,), -1e6, jnp.float32)
        mx, ix = lax.fori_loop(0, PSIZE, _scan, (m0, i0))

        for shift in (1, 2, 4, 8):
            bufm[...] = mx
            bufi[...] = ix
            pidx = lane ^ shift
            mxp = plsc.load_gather(bufm, [pidx])
            ixp = plsc.load_gather(bufi, [pidx])
            take = (mxp > mx) | ((mxp == mx) & (ixp > ix))
            mx = jnp.where(take, mxp, mx)
            ix = jnp.where(take, ixp, ix)
        gidxv = (-ix).astype(jnp.int32) + kshiftv

        pxv = gidxv % PSIZE
        pyv = gidxv // PSIZE
        ox0v = pxv * STRIDE
        oy0v = pyv * STRIDE
        ox1v = jnp.minimum(ox0v + KER - 1, ORISIZE - 1)
        oy1v = jnp.minimum(oy0v + KER - 1, ORISIZE - 1)

        def _patch(rr, carry):
            offv = (oy0v + rr) * ORISIZE + ox0v + lane
            plsc.store_scatter(mbuf, [offv], ones16)
            plsc.store_scatter(mbuf, [offv + 16], ones16)
            return carry
        lax.fori_loop(0, KER, _patch, 0)

        cstage[...] = jnp.where(
            lane == 0, ox0v,
            jnp.where(lane == 1, ox1v,
                      jnp.where(lane == 2, oy0v,
                                jnp.where(lane == 3, oy1v, 0))))
        pltpu.sync_copy(cstage, coord_hbm.at[i, c])
        vstage[...] = jnp.where(lane == 0, mx, jnp.float32(0.0))
        pltpu.sync_copy(vstage, val_hbm.at[i, c])

    pltpu.sync_copy(mbuf, mask_hbm.at[i])


def kernel(infeat, labelTpesudo, labelT, FeatureDA, k):
    del labelTpesudo, FeatureDA
    b = infeat.shape[0]

    pooled = _pooled_grids(infeat)
    pooled_flat = pooled.reshape(b, 2, PFLAT)
    karr = jnp.full((16,), jnp.asarray(k, jnp.float32) - 1.0,
                    dtype=jnp.float32)

    sc = functools.partial(
        pl.kernel,
        mesh=plsc.VectorSubcoreMesh(core_axis_name="c", subcore_axis_name="s"),
        compiler_params=pltpu.CompilerParams(needs_layout_passes=False),
        out_type=[
            jax.ShapeDtypeStruct((b, IMG), jnp.float32),
            jax.ShapeDtypeStruct((b, 2, 16), jnp.int32),
            jax.ShapeDtypeStruct((b, 2, 16), jnp.float32),
        ],
        scratch_types=[
            pltpu.VMEM((PFLAT,), jnp.float32),
            pltpu.VMEM((IMG,), jnp.float32),
            pltpu.VMEM((16,), jnp.float32),
            pltpu.VMEM((16,), jnp.float32),
            pltpu.VMEM((16,), jnp.float32),
            pltpu.VMEM((16,), jnp.int32),
            pltpu.VMEM((16,), jnp.float32),
        ],
    )(_sc_mask_body)
    mask_flat, cbuf, vbuf = sc(pooled_flat, karr)

    mask = mask_flat.reshape(b, 1, ORISIZE, ORISIZE).astype(labelT.dtype)
    coords = jnp.transpose(cbuf[:, :, :4], (1, 0, 2)).reshape(2, b, 2, 2)
    values = jnp.transpose(vbuf[:, :, :1], (1, 0, 2)).reshape(2, b, 1, 1)
    return (mask, coords, values)
